# baseline (speedup 1.0000x reference)
import jax
import jax.numpy as jnp
from jax import lax
from jax.experimental import pallas as pl
from jax.experimental.pallas import tpu as pltpu
from jax.experimental.pallas import tpu_sc as plsc

HIDDEN = 768
NSLICE = HIDDEN // 16
JQ = 4
NJQ = NSLICE // JQ
TOKENS = 8192
B = 4
SEQ = 2048
NUM_TILES = 32
POS_PER_TILE = SEQ // NUM_TILES
PCHUNK = 8
NCHUNK = POS_PER_TILE // PCHUNK
ROWS = B * PCHUNK
EPS = 1e-12


def _body(ids_h, tt_h, wemb_h, pemb_h, temb_h, g_h, b_h, out_h,
          idx_all, tt_all, wbuf0, wbuf1, wbuf2, pbuf0, pbuf1, pbuf2,
          te_v, d_v, g_v, b_v,
          sg0, sg1, sg2, sp0, sp1, sp2, ss0, ss1, ss2):
    c = lax.axis_index("c")
    s = lax.axis_index("s")
    wid = s * 2 + c
    s0 = pl.multiple_of(wid * POS_PER_TILE, POS_PER_TILE)
    tok0 = pl.multiple_of(wid * B * POS_PER_TILE, B * POS_PER_TILE)

    wbuf = (wbuf0, wbuf1, wbuf2)
    pbuf = (pbuf0, pbuf1, pbuf2)
    sg = (sg0, sg1, sg2)
    sp = (sp0, sp1, sp2)
    ss = (ss0, ss1, ss2)

    pltpu.sync_copy(ids_h.at[pl.ds(tok0, B * POS_PER_TILE)], idx_all)
    pltpu.sync_copy(tt_h.at[pl.ds(tok0, B * POS_PER_TILE)], tt_all)
    pltpu.sync_copy(temb_h, te_v)
    pltpu.sync_copy(g_h, g_v)
    pltpu.sync_copy(b_h, b_v)
    for j in range(NSLICE):
        sl = pl.ds(j * 16, 16)
        d_v[sl] = te_v[1, sl] - te_v[0, sl]

    def fire_gather(ck):
        r = ck % 3
        g = pltpu.async_copy(
            wemb_h.at[idx_all.at[pl.ds(ck * ROWS, ROWS)]], wbuf[r], sg[r])
        cs = pl.multiple_of(s0 + ck * PCHUNK, PCHUNK)
        p = pltpu.async_copy(pemb_h.at[pl.ds(cs, PCHUNK)], pbuf[r], sp[r])
        return g, p

    def wait_gather(ck, pend):
        g, p = pend
        g.wait()
        p.wait()

    def fire_scatter(ck):
        r = ck % 3
        cs = s0 + ck * PCHUNK
        return [
            pltpu.async_copy(
                wbuf[r].at[pl.ds(bb * PCHUNK, PCHUNK)],
                out_h.at[pl.ds(pl.multiple_of(bb * SEQ + cs, PCHUNK), PCHUNK)],
                ss[r],
            )
            for bb in range(B)
        ]

    def compute_chunk(ck):
        wb = wbuf[ck % 3]
        pb = pbuf[ck % 3]

        def pos_body(i, carry):
            ttf = [
                plsc.load_gather(
                    tt_all,
                    [jnp.full((16,), ck * ROWS + bb * PCHUNK, jnp.int32) + i],
                ).astype(jnp.float32)
                for bb in range(B)
            ]

            def accum_body(jq, accs):
                acc, acc2 = accs
                acc = list(acc)
                acc2 = list(acc2)
                for jj in range(JQ):
                    sl = pl.ds(jq * (JQ * 16) + jj * 16, 16)
                    pj = pb[i, sl] + te_v[0, sl]
                    dj = d_v[sl]
                    for bb in range(B):
                        r = bb * PCHUNK + i
                        v = wb[r, sl] + pj + ttf[bb] * dj
                        wb[r, sl] = v
                        acc[bb] = acc[bb] + v
                        acc2[bb] = acc2[bb] + v * v
                return tuple(acc), tuple(acc2)

            zero = tuple(jnp.zeros((16,), jnp.float32) for _ in range(B))
            acc, acc2 = lax.fori_loop(0, NJQ, accum_body, (zero, zero))

            meanv = []
            rstdv = []
            for bb in range(B):
                mean = jnp.sum(acc[bb]) * (1.0 / HIDDEN)
                var = jnp.sum(acc2[bb]) * (1.0 / HIDDEN) - mean * mean
                x = jnp.full((16,), var + EPS, jnp.float32)
                xi = lax.bitcast_convert_type(x, jnp.int32)
                yi = 0x5F3759DF - lax.shift_right_logical(xi, 1)
                y = lax.bitcast_convert_type(yi, jnp.float32)
                for _ in range(3):
                    y = y * (1.5 - 0.5 * x * y * y)
                meanv.append(jnp.full((16,), mean, jnp.float32))
                rstdv.append(y)

            def norm_body(jq, carry2):
                for jj in range(JQ):
                    sl = pl.ds(jq * (JQ * 16) + jj * 16, 16)
                    gj = g_v[sl]
                    bj = b_v[sl]
                    for bb in range(B):
                        r = bb * PCHUNK + i
                        v = wb[r, sl]
                        wb[r, sl] = (v - meanv[bb]) * rstdv[bb] * gj + bj
                return carry2

            lax.fori_loop(0, NJQ, norm_body, 0)
            return carry

        lax.fori_loop(0, PCHUNK, pos_body, 0)

    g_pend = [None] * NCHUNK
    s_pend = [None] * NCHUNK
    g_pend[0] = fire_gather(0)
    g_pend[1] = fire_gather(1)
    for ck in range(NCHUNK):
        if 2 <= ck + 1 < NCHUNK:
            if ck - 2 >= 0:
                for h in s_pend[ck - 2]:
                    h.wait()
            g_pend[ck + 1] = fire_gather(ck + 1)
        wait_gather(ck, g_pend[ck])
        compute_chunk(ck)
        s_pend[ck] = fire_scatter(ck)
    for h in s_pend[NCHUNK - 2]:
        h.wait()
    for h in s_pend[NCHUNK - 1]:
        h.wait()


@jax.jit
def kernel(input_ids, token_type_ids, word_emb, pos_emb, type_emb, gamma, beta):
    bsz, seq = input_ids.shape

    def _rearrange(a):
        return (a.reshape(B, NUM_TILES, NCHUNK, PCHUNK)
                 .transpose(1, 2, 0, 3).reshape(-1).astype(jnp.int32))

    ids = _rearrange(input_ids)
    tts = _rearrange(token_type_ids)
    run = pl.kernel(
        _body,
        out_type=jax.ShapeDtypeStruct((TOKENS, HIDDEN), jnp.float32),
        scratch_types=[
            pltpu.VMEM((B * POS_PER_TILE,), jnp.int32),
            pltpu.VMEM((B * POS_PER_TILE,), jnp.int32),
            pltpu.VMEM((ROWS, HIDDEN), jnp.float32),
            pltpu.VMEM((ROWS, HIDDEN), jnp.float32),
            pltpu.VMEM((ROWS, HIDDEN), jnp.float32),
            pltpu.VMEM((PCHUNK, HIDDEN), jnp.float32),
            pltpu.VMEM((PCHUNK, HIDDEN), jnp.float32),
            pltpu.VMEM((PCHUNK, HIDDEN), jnp.float32),
            pltpu.VMEM((2, HIDDEN), jnp.float32),
            pltpu.VMEM((HIDDEN,), jnp.float32),
            pltpu.VMEM((HIDDEN,), jnp.float32),
            pltpu.VMEM((HIDDEN,), jnp.float32),
            pltpu.SemaphoreType.DMA(()),
            pltpu.SemaphoreType.DMA(()),
            pltpu.SemaphoreType.DMA(()),
            pltpu.SemaphoreType.DMA(()),
            pltpu.SemaphoreType.DMA(()),
            pltpu.SemaphoreType.DMA(()),
            pltpu.SemaphoreType.DMA(()),
            pltpu.SemaphoreType.DMA(()),
            pltpu.SemaphoreType.DMA(()),
        ],
        mesh=plsc.VectorSubcoreMesh(core_axis_name="c", subcore_axis_name="s"),
        compiler_params=pltpu.CompilerParams(needs_layout_passes=False),
    )
    out = run(ids, tts, word_emb, pos_emb, type_emb, gamma, beta)
    return out.reshape(bsz, seq, HIDDEN)


# speedup vs baseline: 1.4798x; 1.4798x over previous
import jax
import jax.numpy as jnp
from jax import lax
from jax.experimental import pallas as pl
from jax.experimental.pallas import tpu as pltpu
from jax.experimental.pallas import tpu_sc as plsc

HIDDEN = 768
SEQ = 2048
B = 4
NUM_TILES = 32
GTOK = SEQ // NUM_TILES
EPS = 1e-12
BLK = 512


def _gather_body(ids_h, wemb_h, out_h, idx_v, buf, sem):
    c = lax.axis_index("c")
    s = lax.axis_index("s")
    wid = s * 2 + c
    base = pl.multiple_of(wid * GTOK, GTOK)
    pltpu.sync_copy(ids_h.at[pl.ds(base, GTOK)], idx_v)
    pltpu.async_copy(wemb_h.at[idx_v], buf, sem).wait()
    pltpu.sync_copy(buf, out_h.at[pl.ds(base, GTOK)])


def _sc_gather_slice(ids, word_emb):
    run = pl.kernel(
        _gather_body,
        out_type=jax.ShapeDtypeStruct((SEQ, HIDDEN), jnp.float32),
        scratch_types=[
            pltpu.VMEM((GTOK,), jnp.int32),
            pltpu.VMEM((GTOK, HIDDEN), jnp.float32),
            pltpu.SemaphoreType.DMA(()),
        ],
        mesh=plsc.VectorSubcoreMesh(core_axis_name="c", subcore_axis_name="s"),
        compiler_params=pltpu.CompilerParams(needs_layout_passes=False),
    )
    return run(ids, word_emb)


def _ln_body_first(g_ref, p_ref, tt_ref, te_ref, gm_ref, bt_ref, o_ref):
    _ln_common(g_ref, p_ref, tt_ref, te_ref, gm_ref, bt_ref, o_ref)


def _ln_body_chained(prev_ref, g_ref, p_ref, tt_ref, te_ref, gm_ref, bt_ref,
                     o_ref):
    del prev_ref
    _ln_common(g_ref, p_ref, tt_ref, te_ref, gm_ref, bt_ref, o_ref)


def _ln_common(g_ref, p_ref, tt_ref, te_ref, gm_ref, bt_ref, o_ref):
    tt = tt_ref[...]
    t0 = te_ref[0:1, :]
    t1 = te_ref[1:2, :]
    x = g_ref[...] + p_ref[...] + t0 + tt * (t1 - t0)
    mean = jnp.mean(x, axis=-1, keepdims=True)
    cx = x - mean
    var = jnp.mean(cx * cx, axis=-1, keepdims=True)
    rstd = lax.rsqrt(var + EPS)
    o_ref[0] = cx * rstd * gm_ref[...] + bt_ref[...]


def _tc_ln_slice(prev_out, k, gathered, pos_emb, ttf, type_emb, gamma, beta):
    data_specs = [
        pl.BlockSpec((BLK, HIDDEN), lambda i: (i, 0)),
        pl.BlockSpec((BLK, HIDDEN), lambda i: (i, 0)),
        pl.BlockSpec((BLK, 1), lambda i: (i, 0)),
        pl.BlockSpec((2, HIDDEN), lambda i: (0, 0)),
        pl.BlockSpec((1, HIDDEN), lambda i: (0, 0)),
        pl.BlockSpec((1, HIDDEN), lambda i: (0, 0)),
    ]
    out_spec = pl.BlockSpec((1, BLK, HIDDEN), lambda i: (k, i, 0))
    out_shape = jax.ShapeDtypeStruct((B, SEQ, HIDDEN), jnp.float32)
    args = (gathered, pos_emb, ttf, type_emb, gamma, beta)
    if prev_out is None:
        return pl.pallas_call(
            _ln_body_first,
            grid=(SEQ // BLK,),
            in_specs=data_specs,
            out_specs=out_spec,
            out_shape=out_shape,
        )(*args)
    return pl.pallas_call(
        _ln_body_chained,
        grid=(SEQ // BLK,),
        in_specs=[pl.BlockSpec(memory_space=pl.ANY)] + data_specs,
        out_specs=out_spec,
        out_shape=out_shape,
        input_output_aliases={0: 0},
    )(prev_out, *args)


@jax.jit
def kernel(input_ids, token_type_ids, word_emb, pos_emb, type_emb, gamma, beta):
    bsz, seq = input_ids.shape
    ids = input_ids.astype(jnp.int32)
    ttf = token_type_ids.reshape(bsz, seq, 1).astype(jnp.float32)
    gm = gamma.reshape(1, HIDDEN)
    bt = beta.reshape(1, HIDDEN)
    gathered = [_sc_gather_slice(ids[k], word_emb) for k in range(bsz)]
    out = None
    for k in range(bsz):
        out = _tc_ln_slice(out, k, gathered[k], pos_emb, ttf[k], type_emb,
                           gm, bt)
    return out


# speedup vs baseline: 1.7639x; 1.1920x over previous
import jax
import jax.numpy as jnp
from jax import lax
from jax.experimental import pallas as pl
from jax.experimental.pallas import tpu as pltpu
from jax.experimental.pallas import tpu_sc as plsc

HIDDEN = 768
TOKENS = 8192
NUM_TILES = 32
TOK_PER_TILE = TOKENS // NUM_TILES
CHUNK = 64
EPS = 1e-12
SEQ = 2048
BLK = 512
NBLK = TOKENS // BLK


NRING = 3
GCHUNK = 32
NGCHUNK = TOK_PER_TILE // GCHUNK


def _gather_body(ids_h, wemb_h, out_h,
                 idx_all, buf0, buf1, buf2, sg0, sg1, sg2, ss0, ss1, ss2):
    c = lax.axis_index("c")
    s = lax.axis_index("s")
    wid = s * 2 + c
    base = pl.multiple_of(wid * TOK_PER_TILE, TOK_PER_TILE)

    buf = (buf0, buf1, buf2)
    sg = (sg0, sg1, sg2)
    ss = (ss0, ss1, ss2)

    pltpu.sync_copy(ids_h.at[pl.ds(base, TOK_PER_TILE)], idx_all)

    def fire(ck):
        r = ck % NRING
        return pltpu.async_copy(
            wemb_h.at[idx_all.at[pl.ds(ck * GCHUNK, GCHUNK)]], buf[r], sg[r])

    def scat(ck):
        r = ck % NRING
        off = pl.multiple_of(base + ck * GCHUNK, GCHUNK)
        return pltpu.async_copy(buf[r], out_h.at[pl.ds(off, GCHUNK)], ss[r])

    g_pend = [None] * NGCHUNK
    s_pend = [None] * NGCHUNK
    g_pend[0] = fire(0)
    g_pend[1] = fire(1)
    for ck in range(NGCHUNK):
        if 2 <= ck + 1 < NGCHUNK:
            if ck - 2 >= 0:
                s_pend[ck - 2].wait()
            g_pend[ck + 1] = fire(ck + 1)
        g_pend[ck].wait()
        s_pend[ck] = scat(ck)
    s_pend[NGCHUNK - 2].wait()
    s_pend[NGCHUNK - 1].wait()


def _sc_gather(ids, word_emb):
    run = pl.kernel(
        _gather_body,
        out_type=jax.ShapeDtypeStruct((TOKENS, HIDDEN), jnp.float32),
        scratch_types=[
            pltpu.VMEM((TOK_PER_TILE,), jnp.int32),
            pltpu.VMEM((GCHUNK, HIDDEN), jnp.float32),
            pltpu.VMEM((GCHUNK, HIDDEN), jnp.float32),
            pltpu.VMEM((GCHUNK, HIDDEN), jnp.float32),
            pltpu.SemaphoreType.DMA(()),
            pltpu.SemaphoreType.DMA(()),
            pltpu.SemaphoreType.DMA(()),
            pltpu.SemaphoreType.DMA(()),
            pltpu.SemaphoreType.DMA(()),
            pltpu.SemaphoreType.DMA(()),
        ],
        mesh=plsc.VectorSubcoreMesh(core_axis_name="c", subcore_axis_name="s"),
        compiler_params=pltpu.CompilerParams(needs_layout_passes=False),
    )
    return run(ids, word_emb)


def _ln_body(g_ref, p_ref, tt_ref, te_ref, gm_ref, bt_ref, o_ref):
    tt = tt_ref[...]
    t0 = te_ref[0:1, :][None]
    t1 = te_ref[1:2, :][None]
    x = g_ref[...] + p_ref[...][None] + t0 + tt * (t1 - t0)
    mean = jnp.mean(x, axis=-1, keepdims=True)
    cx = x - mean
    var = jnp.mean(cx * cx, axis=-1, keepdims=True)
    rstd = lax.rsqrt(var + EPS)
    o_ref[...] = cx * rstd * gm_ref[...][None] + bt_ref[...][None]


def _tc_layernorm(gathered, pos_emb, ttf, type_emb, gamma, beta, bsz):
    return pl.pallas_call(
        _ln_body,
        grid=(SEQ // BLK,),
        in_specs=[
            pl.BlockSpec((bsz, BLK, HIDDEN), lambda i: (0, i, 0)),
            pl.BlockSpec((BLK, HIDDEN), lambda i: (i, 0)),
            pl.BlockSpec((bsz, BLK, 1), lambda i: (0, i, 0)),
            pl.BlockSpec((2, HIDDEN), lambda i: (0, 0)),
            pl.BlockSpec((1, HIDDEN), lambda i: (0, 0)),
            pl.BlockSpec((1, HIDDEN), lambda i: (0, 0)),
        ],
        out_specs=pl.BlockSpec((bsz, BLK, HIDDEN), lambda i: (0, i, 0)),
        out_shape=jax.ShapeDtypeStruct((bsz, SEQ, HIDDEN), jnp.float32),
    )(gathered, pos_emb, ttf, type_emb, gamma, beta)


@jax.jit
def kernel(input_ids, token_type_ids, word_emb, pos_emb, type_emb, gamma, beta):
    bsz, seq = input_ids.shape
    ids = input_ids.reshape(-1).astype(jnp.int32)
    ttf = token_type_ids.reshape(bsz, seq, 1).astype(jnp.float32)
    gathered = _sc_gather(ids, word_emb).reshape(bsz, seq, HIDDEN)
    out = _tc_layernorm(gathered, pos_emb, ttf, type_emb,
                        gamma.reshape(1, HIDDEN), beta.reshape(1, HIDDEN), bsz)
    return out
